# Initial kernel scaffold; baseline (speedup 1.0000x reference)
import jax
import jax.numpy as jnp
from jax.experimental import pallas as pl


def kernel(frame, raw, Wb, bb, Wt, bt, Wf, bf, Wg, bg):
    raise NotImplementedError("write your pallas kernel here")


# speedup vs baseline: 2.0762x; 2.0762x over previous
import functools

import jax
import jax.numpy as jnp
from jax.experimental import pallas as pl
from jax.experimental.pallas import tpu as pltpu

B, S, D = 2, 2048, 1024
E = 3


def _fused_kernel(frame_ref, raw_ref, wb_ref, wtf_ref, bb_ref, btf_ref,
                  wg_ref, bg_ref, out_ref):
    frame = frame_ref[...]
    raw = raw_ref[...]
    b = jnp.dot(frame, wb_ref[...], preferred_element_type=jnp.float32)
    b = b + bb_ref[...]
    tf = jnp.dot(raw, wtf_ref[...], preferred_element_type=jnp.float32)
    tf = tf + btf_ref[...]
    t = tf[:, :D]
    f = tf[:, D:]
    wg = wg_ref[...]
    logits = (
        jnp.dot(b, wg[0], preferred_element_type=jnp.float32)
        + jnp.dot(t, wg[1], preferred_element_type=jnp.float32)
        + jnp.dot(f, wg[2], preferred_element_type=jnp.float32)
        + bg_ref[...]
    )
    m = jnp.max(logits, axis=-1, keepdims=True)
    ew = jnp.exp(logits - m)
    w = ew / jnp.sum(ew, axis=-1, keepdims=True)
    out_ref[...] = (
        b * w[:, 0:1] + t * w[:, 1:2] + f * w[:, 2:3]
    )


@functools.partial(jax.jit, static_argnames=("tile",))
def _run(frame2d, raw2d, wb, wtf, bb2d, btf2d, wg3, bg2d, tile=512):
    n_tokens = frame2d.shape[0]
    grid = (n_tokens // tile,)
    return pl.pallas_call(
        _fused_kernel,
        grid=grid,
        in_specs=[
            pl.BlockSpec((tile, D), lambda i: (i, 0)),
            pl.BlockSpec((tile, D), lambda i: (i, 0)),
            pl.BlockSpec((D, D), lambda i: (0, 0)),
            pl.BlockSpec((D, 2 * D), lambda i: (0, 0)),
            pl.BlockSpec((1, D), lambda i: (0, 0)),
            pl.BlockSpec((1, 2 * D), lambda i: (0, 0)),
            pl.BlockSpec((E, D, E), lambda i: (0, 0, 0)),
            pl.BlockSpec((1, E), lambda i: (0, 0)),
        ],
        out_specs=pl.BlockSpec((tile, D), lambda i: (i, 0)),
        out_shape=jax.ShapeDtypeStruct((n_tokens, D), jnp.float32),
        compiler_params=pltpu.CompilerParams(
            dimension_semantics=("arbitrary",),
        ),
    )(frame2d, raw2d, wb, wtf, bb2d, btf2d, wg3, bg2d)


def kernel(frame, raw, Wb, bb, Wt, bt, Wf, bf, Wg, bg):
    frame2d = frame.reshape(B * S, D)
    raw2d = raw.reshape(B * S, D)
    wtf = jnp.concatenate([Wt, Wf], axis=1)
    bb2d = bb.reshape(1, D)
    btf2d = jnp.concatenate([bt, bf]).reshape(1, 2 * D)
    wg3 = Wg.reshape(E, D, E)
    bg2d = bg.reshape(1, E)
    out = _run(frame2d, raw2d, Wb, wtf, bb2d, btf2d, wg3, bg2d)
    return out.reshape(B, S, D)
